# Initial kernel scaffold; baseline (speedup 1.0000x reference)
import jax
import jax.numpy as jnp
from jax.experimental import pallas as pl


def kernel(input, weight):
    raise NotImplementedError("write your pallas kernel here")


# speedup vs baseline: 1.3126x; 1.3126x over previous
import functools

import jax
import jax.numpy as jnp
from jax import lax
from jax.experimental import pallas as pl
from jax.experimental.pallas import tpu as pltpu
from jax.experimental.pallas import tpu_sc as plsc

_TB = 256
_KB = 512
_CH = 128


def _argmin_body(x_ref, sqr_ref, wt_ref, idx_ref):
    k_total = wt_ref.shape[1]
    scores = jnp.dot(x_ref[...], wt_ref[...], preferred_element_type=jnp.float32)
    dist = sqr_ref[...] - 2.0 * scores
    m = jnp.min(dist, axis=1, keepdims=True)
    ids = lax.broadcasted_iota(jnp.int32, dist.shape, 1)
    li = jnp.min(jnp.where(dist == m, ids, k_total), axis=1, keepdims=True)
    idx_ref[...] = li[None]


def _lookup_indices(x2, wt, sqr2, interpret=False):
    n_tok, d = x2.shape
    k_total = wt.shape[1]
    grid = (n_tok // _TB,)
    out = pl.pallas_call(
        _argmin_body,
        grid=grid,
        in_specs=[
            pl.BlockSpec((_TB, d), lambda i: (i, 0)),
            pl.BlockSpec((1, k_total), lambda i: (0, 0)),
            pl.BlockSpec((d, k_total), lambda i: (0, 0)),
        ],
        out_specs=pl.BlockSpec((1, _TB, 1), lambda i: (i, 0, 0)),
        out_shape=jax.ShapeDtypeStruct((n_tok // _TB, _TB, 1), jnp.int32),
        interpret=interpret,
    )(x2, sqr2, wt)
    return out.reshape(-1)


def _gather_rows(table, idx):
    info = plsc.get_sparse_core_info()
    nc, ns = info.num_cores, info.num_subcores
    nw = nc * ns
    n_rows = idx.shape[0]
    d = table.shape[1]
    b_per_w = n_rows // nw
    n_chunks = b_per_w // _CH
    mesh = plsc.VectorSubcoreMesh(core_axis_name="c", subcore_axis_name="s")

    @functools.partial(
        pl.kernel,
        mesh=mesh,
        out_type=jax.ShapeDtypeStruct((n_rows, d), jnp.float32),
        scratch_types=[
            pltpu.VMEM((_CH,), jnp.int32),
            pltpu.VMEM((_CH, d), jnp.float32),
            pltpu.SemaphoreType.DMA,
        ],
    )
    def k(idx_hbm, table_hbm, out_hbm, idx_v, rows_v, sem):
        wid = lax.axis_index("s") * nc + lax.axis_index("c")
        for c in range(n_chunks):
            base = wid * b_per_w + c * _CH
            pltpu.sync_copy(idx_hbm.at[pl.ds(base, _CH)], idx_v)
            pltpu.async_copy(table_hbm.at[idx_v], rows_v, sem).wait()
            pltpu.sync_copy(rows_v, out_hbm.at[pl.ds(base, _CH)])

    return k(idx, table)


def kernel(input, weight):
    b, t, d = input.shape
    x2 = input.reshape(b * t, d).astype(jnp.bfloat16)
    wt = weight.T.astype(jnp.bfloat16)
    sqr2 = jnp.sum(weight * weight, axis=1)[None, :]
    idx = _lookup_indices(x2, wt, sqr2)
    vec2 = _gather_rows(weight, idx)
    vectors = vec2.reshape(b, t, d)
    indices = idx.reshape(b, t)
    values = input + lax.stop_gradient(vectors - input)
    return (values, indices, vectors)
